# baseline (speedup 1.0000x reference)
import functools

import jax
import jax.numpy as jnp
from jax import lax
from jax.experimental import pallas as pl
from jax.experimental.pallas import tpu as pltpu
from jax.experimental.pallas import tpu_sc as plsc

N, K, D = 10000, 32, 128


BN_MM = 2000


def _tail_body(xin_ref, w_ref, b_ref, out_ref):
    out_ref[...] = (
        jnp.dot(xin_ref[...], w_ref[...], preferred_element_type=jnp.float32)
        + b_ref[...]
    )


def _tail_matmul(e_t_input, w_t, b_row):
    grid = (N // BN_MM,)
    return pl.pallas_call(
        _tail_body,
        grid=grid,
        in_specs=[
            pl.BlockSpec((BN_MM, D), lambda i: (i, 0)),
            pl.BlockSpec((D, D), lambda i: (0, 0)),
            pl.BlockSpec((1, D), lambda i: (0, 0)),
        ],
        out_specs=pl.BlockSpec((BN_MM, D), lambda i: (i, 0)),
        out_shape=jax.ShapeDtypeStruct((N, D), jnp.float32),
    )(e_t_input, w_t, b_row)


NW = 32
NBUF = 5


def _make_gather_body(nk, ch):
    idx_per_w = nk // NW
    nch = idx_per_w // ch

    def _gather_body(e_t_hbm, idx_hbm, out_hbm, idx_v, rows, gsems, ssems):
        info = plsc.get_sparse_core_info()
        wid = lax.axis_index("s") * info.num_cores + lax.axis_index("c")
        base = wid * idx_per_w
        pltpu.sync_copy(idx_hbm.at[wid], idx_v)

        def gather(c, b):
            return pltpu.make_async_copy(
                e_t_hbm.at[idx_v.at[c]], rows[b], gsems[b])

        def scatter(c, b):
            return pltpu.make_async_copy(
                rows[b], out_hbm.at[pl.ds(base + c * ch, ch)], ssems[b])

        for b in range(NBUF):
            gather(b, b).start()

        def step(i, carry):
            c = NBUF * i
            for b in range(NBUF):
                gather(c + b, b).wait()
                scatter(c + b, b).start()
            for b in range(NBUF):
                scatter(c + b, b).wait()

                @pl.when(i < nch // NBUF - 1)
                def _():
                    gather(c + NBUF + b, b).start()

            return carry

        lax.fori_loop(0, nch // NBUF, step, 0)

    return _gather_body


def _sc_gather(e_t, idx_flat, nk, ch):
    idx_per_w = nk // NW
    nch = idx_per_w // ch
    mesh = plsc.VectorSubcoreMesh(core_axis_name="c", subcore_axis_name="s")
    return pl.kernel(
        _make_gather_body(nk, ch),
        out_type=jax.ShapeDtypeStruct((nk, D), jnp.float32),
        mesh=mesh,
        scratch_types=[
            pltpu.VMEM((nch, ch), jnp.int32),
            [pltpu.VMEM((ch, D), jnp.float32) for _ in range(NBUF)],
            [pltpu.SemaphoreType.DMA for _ in range(NBUF)],
            [pltpu.SemaphoreType.DMA for _ in range(NBUF)],
        ],
    )(e_t, idx_flat.reshape(NW, nch, ch))


BN = 400


def _attn_body(ehin_ref, nb_ref, tw_ref, x_ref, wh_ref, bh_ref,
               w1_ref, b1_ref, w2_ref, b2_ref, g_ref, bb_ref, out_ref):
    eh = (
        jnp.dot(ehin_ref[...], wh_ref[...], preferred_element_type=jnp.float32)
        + bh_ref[...]
    )
    bn = nb_ref.shape[0]
    nb = nb_ref[...]
    w = tw_ref[...][:, :, None]
    gate = jnp.tanh((2.0 - w) * eh[:, None, :] + w * nb)
    ones_mat = jnp.ones((D, D), jnp.float32)
    ka = jnp.dot(
        (nb * gate).reshape(bn * K, D), ones_mat,
        preferred_element_type=jnp.float32,
    ).reshape(bn, K, D)
    u = jnp.exp(ka - jnp.max(ka, axis=1, keepdims=True))
    numer = jnp.sum(u * nb, axis=1)
    denom = jnp.sum(u, axis=1)
    e_nh = numer / denom
    x = x_ref[...]
    s = (
        jnp.dot(x + e_nh, w1_ref[...], preferred_element_type=jnp.float32)
        + b1_ref[...]
    )
    s = jnp.where(s >= 0.0, s, 0.01 * s)
    bi = (
        jnp.dot(x * e_nh, w2_ref[...], preferred_element_type=jnp.float32)
        + b2_ref[...]
    )
    bi = jnp.where(bi >= 0.0, bi, 0.01 * bi)
    h = s + bi
    mu = jnp.mean(h, axis=-1, keepdims=True)
    var = jnp.mean((h - mu) ** 2, axis=-1, keepdims=True)
    out_ref[...] = (h - mu) * lax.rsqrt(var + 1e-5) * g_ref[...] + bb_ref[...]


def _attn_mlp(e_h_input, nb3, tw, x, wh_t, bh, w1_t, b1, w2_t, b2, g, bb,
              ng, bn):
    grid = (ng // bn,)
    full = lambda i: (0, 0)
    return pl.pallas_call(
        _attn_body,
        grid=grid,
        in_specs=[
            pl.BlockSpec((bn, D), lambda i: (i, 0)),
            pl.BlockSpec((bn, K, D), lambda i: (i, 0, 0)),
            pl.BlockSpec((bn, K), lambda i: (i, 0)),
            pl.BlockSpec((bn, D), lambda i: (i, 0)),
            pl.BlockSpec((D, D), full),
            pl.BlockSpec((1, D), full),
            pl.BlockSpec((D, D), full),
            pl.BlockSpec((1, D), full),
            pl.BlockSpec((D, D), full),
            pl.BlockSpec((1, D), full),
            pl.BlockSpec((1, D), full),
            pl.BlockSpec((1, D), full),
        ],
        out_specs=pl.BlockSpec((bn, D), lambda i: (i, 0)),
        out_shape=jax.ShapeDtypeStruct((ng, D), jnp.float32),
    )(e_h_input, nb3, tw, x, wh_t, bh, w1_t, b1, w2_t, b2, g, bb)


PHASES = (5000, 3000, 2000)
CH = 40
BN_P = 1000


def kernel(x, e_h_input, e_t_input, topk_indices, topk_weights,
           W_head_w, W_head_b, W_tail_w, W_tail_b,
           lin1_w, lin1_b, lin2_w, lin2_b, ln_g, ln_b):
    e_t = _tail_matmul(e_t_input, W_tail_w.T, W_tail_b.reshape(1, D))
    wh_t = W_head_w.T
    bh = W_head_b.reshape(1, D)
    w1_t = lin1_w.T
    b1 = lin1_b.reshape(1, D)
    w2_t = lin2_w.T
    b2 = lin2_b.reshape(1, D)
    g = ln_g.reshape(1, D)
    bb = ln_b.reshape(1, D)
    outs = []
    lo = 0
    for ng in PHASES:
        idx = lax.slice_in_dim(topk_indices, lo, lo + ng).reshape(ng * K)
        nb3 = _sc_gather(e_t, idx, ng * K, CH).reshape(ng, K, D)
        outs.append(_attn_mlp(
            lax.slice_in_dim(e_h_input, lo, lo + ng), nb3,
            lax.slice_in_dim(topk_weights, lo, lo + ng),
            lax.slice_in_dim(x, lo, lo + ng),
            wh_t, bh, w1_t, b1, w2_t, b2, g, bb, ng, BN_P))
        lo += ng
    return jnp.concatenate(outs, axis=0)


# speedup vs baseline: 1.0528x; 1.0528x over previous
import jax
import jax.numpy as jnp
from jax import lax
from jax.experimental import pallas as pl
from jax.experimental.pallas import tpu as pltpu
from jax.experimental.pallas import tpu_sc as plsc

N, K, D = 10000, 32, 128


BN_MM = 2000


def _tail_body(xin_ref, w_ref, b_ref, out_ref):
    out_ref[...] = (
        jnp.dot(xin_ref[...], w_ref[...], preferred_element_type=jnp.float32)
        + b_ref[...]
    )


def _tail_matmul(e_t_input, w_t, b_row):
    grid = (N // BN_MM,)
    return pl.pallas_call(
        _tail_body,
        grid=grid,
        in_specs=[
            pl.BlockSpec((BN_MM, D), lambda i: (i, 0)),
            pl.BlockSpec((D, D), lambda i: (0, 0)),
            pl.BlockSpec((1, D), lambda i: (0, 0)),
        ],
        out_specs=pl.BlockSpec((BN_MM, D), lambda i: (i, 0)),
        out_shape=jax.ShapeDtypeStruct((N, D), jnp.float32),
    )(e_t_input, w_t, b_row)


NW = 32
NBUF = 5


def _make_gather_body(nk, ch):
    idx_per_w = nk // NW
    nch = idx_per_w // ch

    def _gather_body(e_t_hbm, idx_hbm, out_hbm, idx_v, rows, gsems, ssems):
        info = plsc.get_sparse_core_info()
        wid = lax.axis_index("s") * info.num_cores + lax.axis_index("c")
        base = wid * idx_per_w
        pltpu.sync_copy(idx_hbm.at[wid], idx_v)

        def gather(c, b):
            return pltpu.make_async_copy(
                e_t_hbm.at[idx_v.at[c]], rows[b], gsems[b])

        def scatter(c, b):
            return pltpu.make_async_copy(
                rows[b], out_hbm.at[pl.ds(base + c * ch, ch)], ssems[b])

        for b in range(NBUF):
            gather(b, b).start()

        def step(i, carry):
            c = NBUF * i
            for b in range(NBUF):
                gather(c + b, b).wait()
                scatter(c + b, b).start()
            for b in range(NBUF):
                scatter(c + b, b).wait()

                @pl.when(i < nch // NBUF - 1)
                def _():
                    gather(c + NBUF + b, b).start()

            return carry

        lax.fori_loop(0, nch // NBUF, step, 0)

    return _gather_body


def _sc_gather(e_t, idx_flat, nk, ch):
    idx_per_w = nk // NW
    nch = idx_per_w // ch
    mesh = plsc.VectorSubcoreMesh(core_axis_name="c", subcore_axis_name="s")
    return pl.kernel(
        _make_gather_body(nk, ch),
        out_type=jax.ShapeDtypeStruct((nk, D), jnp.float32),
        mesh=mesh,
        scratch_types=[
            pltpu.VMEM((nch, ch), jnp.int32),
            [pltpu.VMEM((ch, D), jnp.float32) for _ in range(NBUF)],
            [pltpu.SemaphoreType.DMA for _ in range(NBUF)],
            [pltpu.SemaphoreType.DMA for _ in range(NBUF)],
        ],
    )(e_t, idx_flat.reshape(NW, nch, ch))


def _attn_body(ehin_ref, nb_ref, tw_ref, x_ref, wh_ref, bh_ref,
               w1_ref, b1_ref, w2_ref, b2_ref, g_ref, bb_ref, out_ref):
    eh = (
        jnp.dot(ehin_ref[...], wh_ref[...], preferred_element_type=jnp.float32)
        + bh_ref[...]
    )
    bn = nb_ref.shape[0]
    nb = nb_ref[...]
    w = tw_ref[...][:, :, None]
    gate = jnp.tanh((2.0 - w) * eh[:, None, :] + w * nb)
    ones_mat = jnp.ones((D, D), jnp.float32)
    ka = jnp.dot(
        (nb * gate).reshape(bn * K, D), ones_mat,
        preferred_element_type=jnp.float32,
    ).reshape(bn, K, D)
    u = jnp.exp(ka - jnp.max(ka, axis=1, keepdims=True))
    numer = jnp.sum(u * nb, axis=1)
    denom = jnp.sum(u, axis=1)
    e_nh = numer / denom
    x = x_ref[...]
    s = (
        jnp.dot(x + e_nh, w1_ref[...], preferred_element_type=jnp.float32)
        + b1_ref[...]
    )
    s = jnp.where(s >= 0.0, s, 0.01 * s)
    bi = (
        jnp.dot(x * e_nh, w2_ref[...], preferred_element_type=jnp.float32)
        + b2_ref[...]
    )
    bi = jnp.where(bi >= 0.0, bi, 0.01 * bi)
    h = s + bi
    mu = jnp.mean(h, axis=-1, keepdims=True)
    var = jnp.mean((h - mu) ** 2, axis=-1, keepdims=True)
    out_ref[...] = (h - mu) * lax.rsqrt(var + 1e-5) * g_ref[...] + bb_ref[...]


def _attn_mlp(e_h_input, nb3, tw, x, wh_t, bh, w1_t, b1, w2_t, b2, g, bb,
              ng, bn):
    grid = (ng // bn,)
    full = lambda i: (0, 0)
    return pl.pallas_call(
        _attn_body,
        grid=grid,
        in_specs=[
            pl.BlockSpec((bn, D), lambda i: (i, 0)),
            pl.BlockSpec((bn, K, D), lambda i: (i, 0, 0)),
            pl.BlockSpec((bn, K), lambda i: (i, 0)),
            pl.BlockSpec((bn, D), lambda i: (i, 0)),
            pl.BlockSpec((D, D), full),
            pl.BlockSpec((1, D), full),
            pl.BlockSpec((D, D), full),
            pl.BlockSpec((1, D), full),
            pl.BlockSpec((D, D), full),
            pl.BlockSpec((1, D), full),
            pl.BlockSpec((1, D), full),
            pl.BlockSpec((1, D), full),
        ],
        out_specs=pl.BlockSpec((bn, D), lambda i: (i, 0)),
        out_shape=jax.ShapeDtypeStruct((ng, D), jnp.float32),
    )(e_h_input, nb3, tw, x, wh_t, bh, w1_t, b1, w2_t, b2, g, bb)


PHASES = (5000, 5000)
CH = 40
BN_P = 1000


def kernel(x, e_h_input, e_t_input, topk_indices, topk_weights,
           W_head_w, W_head_b, W_tail_w, W_tail_b,
           lin1_w, lin1_b, lin2_w, lin2_b, ln_g, ln_b):
    e_t = _tail_matmul(e_t_input, W_tail_w.T, W_tail_b.reshape(1, D))
    wh_t = W_head_w.T
    bh = W_head_b.reshape(1, D)
    w1_t = lin1_w.T
    b1 = lin1_b.reshape(1, D)
    w2_t = lin2_w.T
    b2 = lin2_b.reshape(1, D)
    g = ln_g.reshape(1, D)
    bb = ln_b.reshape(1, D)
    outs = []
    lo = 0
    for ng in PHASES:
        idx = lax.slice_in_dim(topk_indices, lo, lo + ng).reshape(ng * K)
        nb3 = _sc_gather(e_t, idx, ng * K, CH).reshape(ng, K, D)
        outs.append(_attn_mlp(
            lax.slice_in_dim(e_h_input, lo, lo + ng), nb3,
            lax.slice_in_dim(topk_weights, lo, lo + ng),
            lax.slice_in_dim(x, lo, lo + ng),
            wh_t, bh, w1_t, b1, w2_t, b2, g, bb, ng, BN_P))
        lo += ng
    return jnp.concatenate(outs, axis=0)
